# baseline (speedup 1.0000x reference)
import functools

import jax
import jax.numpy as jnp
from jax import lax
from jax.experimental import pallas as pl
from jax.experimental.pallas import tpu as pltpu
from jax.experimental.pallas import tpu_sc as plsc

D = 64
NBUF = 4


@functools.cache
def _make_gather(bsz, tsz):
    info = plsc.get_sparse_core_info()
    nw = info.num_cores * info.num_subcores
    b_per_w = bsz // nw
    assert b_per_w * nw == bsz and b_per_w % NBUF == 0 and tsz <= 128

    mesh = plsc.VectorSubcoreMesh(core_axis_name="c", subcore_axis_name="s")

    @functools.partial(
        pl.kernel,
        mesh=mesh,
        out_type=jax.ShapeDtypeStruct((bsz, tsz, D), jnp.float32),
        scratch_types=[
            pltpu.VMEM((b_per_w, tsz), jnp.int32),
            pltpu.VMEM((NBUF, tsz, D), jnp.float32),
            pltpu.SemaphoreType.DMA,
            pltpu.SemaphoreType.DMA,
            pltpu.SemaphoreType.DMA,
            pltpu.SemaphoreType.DMA,
            pltpu.SemaphoreType.DMA,
            pltpu.SemaphoreType.DMA,
            pltpu.SemaphoreType.DMA,
            pltpu.SemaphoreType.DMA,
        ],
        compiler_params=pltpu.CompilerParams(use_tc_tiling_on_sc=False),
    )
    def gather(idx_hbm, table_hbm, out_hbm, idx_v, rows_v, *sems):
        gsems = sems[:NBUF]
        ssems = sems[NBUF:]
        wid = lax.axis_index("s") * info.num_cores + lax.axis_index("c")
        pltpu.sync_copy(idx_hbm.at[wid], idx_v)
        base = wid * b_per_w

        def fire_gather(g, b):
            pltpu.async_copy(
                table_hbm.at[idx_v.at[g]], rows_v.at[b], gsems[b]
            )

        def drain_gather(g, b):
            pltpu.make_async_copy(
                table_hbm.at[idx_v.at[g]], rows_v.at[b], gsems[b]
            ).wait()

        def fire_store(g, b):
            pltpu.async_copy(rows_v.at[b], out_hbm.at[base + g], ssems[b])

        def drain_store(g, b):
            pltpu.make_async_copy(
                rows_v.at[b], out_hbm.at[base + g], ssems[b]
            ).wait()

        fire_gather(0, 0)
        fire_gather(1, 1)

        @pl.loop(0, b_per_w, step=NBUF)
        def _(sg):
            for b in range(NBUF):
                s = sg + b
                drain_gather(s, b)
                if b >= 2:
                    drain_store(s - 2, (b + 2) % NBUF)
                else:

                    @pl.when(sg >= 2)
                    def _():
                        drain_store(s - 2, (b + 2) % NBUF)

                fire_store(s, b)
                if b < 2:
                    fire_gather(s + 2, (b + 2) % NBUF)
                else:

                    @pl.when(s + 2 < b_per_w)
                    def _():
                        fire_gather(s + 2, (b + 2) % NBUF)

        drain_store(b_per_w - 2, (NBUF - 2) % NBUF)
        drain_store(b_per_w - 1, NBUF - 1)

    return gather


@jax.jit
def kernel(idx, weight):
    b, t = idx.shape
    info = plsc.get_sparse_core_info()
    nw = info.num_cores * info.num_subcores
    idx3 = idx.reshape(nw, b // nw, t).astype(jnp.int32)
    return _make_gather(b, t)(idx3, weight)


# speedup vs baseline: 1.0282x; 1.0282x over previous
import functools

import jax
import jax.numpy as jnp
from jax import lax
from jax.experimental import pallas as pl
from jax.experimental.pallas import tpu as pltpu
from jax.experimental.pallas import tpu_sc as plsc

D = 64
CHUNK = 128
K = 5
NBUF = 2


@functools.cache
def _make_gather(n_rows):
    info = plsc.get_sparse_core_info()
    nw = info.num_cores * info.num_subcores
    rows_per_w = n_rows // nw
    n_chunks = rows_per_w // CHUNK
    n_super = n_chunks // K
    s_rows = K * CHUNK
    assert n_super * K == n_chunks and n_chunks * CHUNK * nw == n_rows
    assert n_super % NBUF == 0 and n_super >= 2 * NBUF

    mesh = plsc.VectorSubcoreMesh(core_axis_name="c", subcore_axis_name="s")

    @functools.partial(
        pl.kernel,
        mesh=mesh,
        out_type=jax.ShapeDtypeStruct((n_rows, D), jnp.float32),
        scratch_types=[
            pltpu.VMEM((n_chunks, CHUNK), jnp.int32),
            pltpu.VMEM((NBUF, s_rows, D), jnp.float32),
            pltpu.SemaphoreType.DMA,
            pltpu.SemaphoreType.DMA,
            pltpu.SemaphoreType.DMA,
            pltpu.SemaphoreType.DMA,
        ],
        compiler_params=pltpu.CompilerParams(use_tc_tiling_on_sc=False),
    )
    def gather(idx_hbm, table_hbm, out_hbm, idx_v, rows_v, g0, g1, s0_, s1_):
        gsems = (g0, g1)
        ssems = (s0_, s1_)
        wid = lax.axis_index("s") * info.num_cores + lax.axis_index("c")
        pltpu.sync_copy(idx_hbm.at[wid], idx_v)
        base = wid * rows_per_w

        def fire_gathers(s, b):
            for j in range(K):
                pltpu.async_copy(
                    table_hbm.at[idx_v.at[s * K + j]],
                    rows_v.at[b].at[pl.ds(j * CHUNK, CHUNK)],
                    gsems[b],
                )

        def drain_gathers(s, b):
            for j in range(K):
                pltpu.make_async_copy(
                    table_hbm.at[idx_v.at[s * K + j]],
                    rows_v.at[b].at[pl.ds(j * CHUNK, CHUNK)],
                    gsems[b],
                ).wait()

        def fire_store(s, b):
            row0 = pl.multiple_of(base + s * s_rows, CHUNK)
            pltpu.async_copy(rows_v.at[b], out_hbm.at[pl.ds(row0, s_rows)], ssems[b])

        def drain_store(s, b):
            row0 = pl.multiple_of(base + s * s_rows, CHUNK)
            pltpu.make_async_copy(
                rows_v.at[b], out_hbm.at[pl.ds(row0, s_rows)], ssems[b]
            ).wait()

        fire_gathers(0, 0)

        @pl.loop(0, n_super, step=NBUF)
        def _(sg):
            for b in range(NBUF):
                s = sg + b
                drain_gathers(s, b)
                if b == 1:
                    drain_store(s - 1, b ^ 1)
                else:

                    @pl.when(sg >= 1)
                    def _():
                        drain_store(s - 1, b ^ 1)

                fire_store(s, b)
                if b == 0:
                    fire_gathers(s + 1, b ^ 1)
                else:

                    @pl.when(s + 1 < n_super)
                    def _():
                        fire_gathers(s + 1, b ^ 1)

        drain_store(n_super - 1, 1)

    return gather


@jax.jit
def kernel(idx, weight):
    b, t = idx.shape
    n = b * t
    info = plsc.get_sparse_core_info()
    nw = info.num_cores * info.num_subcores
    idx3 = idx.reshape(nw, n // (nw * CHUNK), CHUNK).astype(jnp.int32)
    out = _make_gather(n)(idx3, weight)
    return out.reshape(b, t, weight.shape[1])
